# baseline (speedup 1.0000x reference)
import functools

import jax
import jax.experimental.layout
import jax.numpy as jnp
from jax import lax
from jax.experimental import pallas as pl
from jax.experimental.pallas import tpu as pltpu
from jax.experimental.pallas import tpu_sc as plsc

_N_FIELDS = 26
_EMBED_DIM = 64
_BATCH = 4096
_TOTAL = _BATCH * _N_FIELDS
_NC, _NS, _LANES = 2, 16, 16
_NW = _NC * _NS
_SAMP_W = _BATCH // _NW
_PER_W = _SAMP_W * _N_FIELDS
_BG = _SAMP_W // _LANES
_NB = 3

_mesh = plsc.VectorSubcoreMesh(core_axis_name="c", subcore_axis_name="s")


@functools.partial(
    pl.kernel,
    mesh=_mesh,
    out_type=jax.ShapeDtypeStruct(
        (_N_FIELDS, _EMBED_DIM // 8, _BATCH // 128, 8, 128), jnp.float32),
    scratch_types=[
        pltpu.VMEM((_PER_W,), jnp.int32),
        pltpu.VMEM((_N_FIELDS, _SAMP_W), jnp.int32),
        pltpu.VMEM((_NB, _SAMP_W, _EMBED_DIM), jnp.float32),
        pltpu.VMEM((_NB, _EMBED_DIM, _SAMP_W), jnp.float32),
        [pltpu.SemaphoreType.DMA] * _NB,
        [pltpu.SemaphoreType.DMA] * _NB,
    ],
    compiler_params=pltpu.CompilerParams(
        use_tc_tiling_on_sc=False, needs_layout_passes=False),
)
def _emb_lookup(x_hbm, table_hbm, out_hbm, xv, pfi, praw, tbuf, gsems, wsems):
    wid = lax.axis_index("s") * _NC + lax.axis_index("c")

    pltpu.sync_copy(x_hbm.at[pl.ds(wid * _PER_W, _PER_W)], xv)

    lane = lax.iota(jnp.int32, _LANES)
    lane26 = lane * _N_FIELDS
    bvec = [lane + bg * _LANES for bg in range(_BG)]

    for f in range(_N_FIELDS):
        row = pfi.at[f]
        for bg in range(_BG):
            v = plsc.load_gather(xv, [lane26 + (bg * _LANES * _N_FIELDS + f)])
            row[pl.ds(bg * _LANES, _LANES)] = v + f * 1000

    def gather(f, b):
        return pltpu.async_copy(table_hbm.at[pfi.at[f]], praw.at[b], gsems[b])

    def transpose(b):
        src = praw.at[b]
        dst = tbuf.at[b]

        def body(d, _):
            dvec = lax.broadcast(d, (_LANES,))
            drow = dst.at[d]
            for bg in range(_BG):
                drow[pl.ds(bg * _LANES, _LANES)] = plsc.load_gather(
                    src, [bvec[bg], dvec])
            return 0

        lax.fori_loop(0, _EMBED_DIM, body, 0)

    def write(f, b):
        return [
            pltpu.async_copy(
                tbuf.at[b].at[pl.ds(db * 8, 8)],
                out_hbm.at[f, db, wid],
                wsems[b])
            for db in range(_EMBED_DIM // 8)
        ]

    hg = [None] * _NB
    hw = [[] for _ in range(_NB)]
    for b in range(_NB):
        hg[b] = gather(b, b)
    for f in range(_N_FIELDS):
        b = f % _NB
        hg[b].wait()
        for h in hw[b]:
            h.wait()
        transpose(b)
        hw[b] = write(f, b)
        nxt = f + _NB
        if nxt < _N_FIELDS:
            hg[b] = gather(nxt, b)
    for f in range(_N_FIELDS - _NB, _N_FIELDS):
        for h in hw[f % _NB]:
            h.wait()


def kernel(x, table):
    p = _emb_lookup(x.reshape(_TOTAL), table)
    out = p.transpose(2, 4, 0, 1, 3).reshape(_BATCH, _N_FIELDS, _EMBED_DIM)
    return jax.experimental.layout.with_layout_constraint(
        out, jax.experimental.layout.Layout(major_to_minor=(1, 2, 0)))


# speedup vs baseline: 1.5804x; 1.5804x over previous
import functools

import jax
import jax.numpy as jnp
from jax import lax
from jax.experimental import pallas as pl
from jax.experimental.pallas import tpu as pltpu
from jax.experimental.pallas import tpu_sc as plsc

_N_FIELDS = 26
_EMBED_DIM = 64
_BATCH = 4096
_TOTAL = _BATCH * _N_FIELDS
_NC, _NS, _LANES = 2, 16, 16
_NW = _NC * _NS
_PER_W = _TOTAL // _NW
_ROW = 128
_G = _PER_W // _ROW
_NB = 4

_mesh = plsc.VectorSubcoreMesh(core_axis_name="c", subcore_axis_name="s")


@functools.partial(
    pl.kernel,
    mesh=_mesh,
    out_type=jax.ShapeDtypeStruct((_TOTAL, _EMBED_DIM), jnp.float32),
    scratch_types=[
        pltpu.VMEM((_PER_W,), jnp.int32),
        pltpu.VMEM((_NB, _ROW, _EMBED_DIM), jnp.float32),
        [pltpu.SemaphoreType.DMA] * _NB,
        [pltpu.SemaphoreType.DMA] * _NB,
    ],
    compiler_params=pltpu.CompilerParams(use_tc_tiling_on_sc=False),
)
def _emb_lookup(x_hbm, table_hbm, out_hbm, idx_v, rows_v, gsems, wsems):
    wid = lax.axis_index("s") * _NC + lax.axis_index("c")
    base = wid * _PER_W

    lane = lax.iota(jnp.int32, _LANES)
    pats = {
        s: ((s + lane) % _N_FIELDS) * 1000 for s in range(0, _N_FIELDS, 2)
    }

    pltpu.sync_copy(x_hbm.at[pl.ds(base, _PER_W)], idx_v)
    for j in range(_PER_W // _LANES):
        s = pl.ds(j * _LANES, _LANES)
        idx_v[s] = idx_v[s] + pats[(j * _LANES) % _N_FIELDS]

    def gather(g, b):
        return pltpu.async_copy(
            table_hbm.at[idx_v.at[pl.ds(g * _ROW, _ROW)]], rows_v.at[b],
            gsems[b])

    def write(g, b):
        return pltpu.async_copy(
            rows_v.at[b], out_hbm.at[pl.ds(base + g * _ROW, _ROW)], wsems[b])

    hg = [None] * _NB
    hw = [None] * _NB
    for b in range(_NB):
        hg[b] = gather(b, b)
    for g in range(_G):
        b = g % _NB
        hg[b].wait()
        hw[b] = write(g, b)
        nxt = g + _NB
        if nxt < _G:
            hw[b].wait()
            hg[b] = gather(nxt, b)
    for g in range(max(0, _G - _NB), _G):
        hw[g % _NB].wait()


def kernel(x, table):
    out = _emb_lookup(x.reshape(_TOTAL), table)
    return out.reshape(_BATCH, _N_FIELDS, _EMBED_DIM)


# speedup vs baseline: 1.5862x; 1.0037x over previous
import functools

import jax
import jax.numpy as jnp
from jax import lax
from jax.experimental import pallas as pl
from jax.experimental.pallas import tpu as pltpu
from jax.experimental.pallas import tpu_sc as plsc

_N_FIELDS = 26
_EMBED_DIM = 64
_BATCH = 4096
_TOTAL = _BATCH * _N_FIELDS
_NC, _NS, _LANES = 2, 16, 16
_NW = _NC * _NS
_PER_W = _TOTAL // _NW
_ROW = 128
_G = _PER_W // _ROW
_NB = 6

_mesh = plsc.VectorSubcoreMesh(core_axis_name="c", subcore_axis_name="s")


@functools.partial(
    pl.kernel,
    mesh=_mesh,
    out_type=jax.ShapeDtypeStruct((_TOTAL, _EMBED_DIM), jnp.float32),
    scratch_types=[
        pltpu.VMEM((_PER_W,), jnp.int32),
        pltpu.VMEM((_NB, _ROW, _EMBED_DIM), jnp.float32),
        [pltpu.SemaphoreType.DMA] * _NB,
        [pltpu.SemaphoreType.DMA] * _NB,
    ],
    compiler_params=pltpu.CompilerParams(use_tc_tiling_on_sc=False),
)
def _emb_lookup(x_hbm, table_hbm, out_hbm, idx_v, rows_v, gsems, wsems):
    wid = lax.axis_index("s") * _NC + lax.axis_index("c")
    base = wid * _PER_W

    lane = lax.iota(jnp.int32, _LANES)
    pats = {
        s: ((s + lane) % _N_FIELDS) * 1000 for s in range(0, _N_FIELDS, 2)
    }

    pltpu.sync_copy(x_hbm.at[pl.ds(base, _PER_W)], idx_v)
    for j in range(_PER_W // _LANES):
        s = pl.ds(j * _LANES, _LANES)
        idx_v[s] = idx_v[s] + pats[(j * _LANES) % _N_FIELDS]

    def gather(g, b):
        return pltpu.async_copy(
            table_hbm.at[idx_v.at[pl.ds(g * _ROW, _ROW)]], rows_v.at[b],
            gsems[b])

    def write(g, b):
        return pltpu.async_copy(
            rows_v.at[b], out_hbm.at[pl.ds(base + g * _ROW, _ROW)], wsems[b])

    hg = [None] * _NB
    hw = [None] * _NB
    for b in range(_NB):
        hg[b] = gather(b, b)
    for g in range(_G):
        b = g % _NB
        hg[b].wait()
        hw[b] = write(g, b)
        nxt = g + _NB
        if nxt < _G:
            hw[b].wait()
            hg[b] = gather(nxt, b)
    for g in range(max(0, _G - _NB), _G):
        hw[g % _NB].wait()


def kernel(x, table):
    out = _emb_lookup(x.reshape(_TOTAL), table)
    return out.reshape(_BATCH, _N_FIELDS, _EMBED_DIM)


# speedup vs baseline: 1.5901x; 1.0025x over previous
import functools

import jax
import jax.numpy as jnp
from jax import lax
from jax.experimental import pallas as pl
from jax.experimental.pallas import tpu as pltpu
from jax.experimental.pallas import tpu_sc as plsc

_N_FIELDS = 26
_EMBED_DIM = 64
_BATCH = 4096
_TOTAL = _BATCH * _N_FIELDS
_NC, _NS, _LANES = 2, 16, 16
_NW = _NC * _NS
_PER_W = _TOTAL // _NW
_ROW = 128
_G = _PER_W // _ROW
_NB = 6

_mesh = plsc.VectorSubcoreMesh(core_axis_name="c", subcore_axis_name="s")


@functools.partial(
    pl.kernel,
    mesh=_mesh,
    out_type=jax.ShapeDtypeStruct((_TOTAL, _EMBED_DIM), jnp.float32),
    scratch_types=[
        pltpu.VMEM((_PER_W,), jnp.int32),
        pltpu.VMEM((_NB, _ROW, _EMBED_DIM), jnp.float32),
        [pltpu.SemaphoreType.DMA] * _NB,
        [pltpu.SemaphoreType.DMA] * _NB,
    ],
    compiler_params=pltpu.CompilerParams(use_tc_tiling_on_sc=False),
)
def _emb_lookup(x_hbm, table_hbm, out_hbm, idx_v, rows_v, gsems, wsems):
    wid = lax.axis_index("s") * _NC + lax.axis_index("c")
    base = wid * _PER_W

    lane = lax.iota(jnp.int32, _LANES)
    pats = {
        s: ((s + lane) % _N_FIELDS) * 1000 for s in range(0, _N_FIELDS, 2)
    }

    pltpu.sync_copy(x_hbm.at[pl.ds(base, _PER_W)], idx_v)

    def add_offsets(g):
        for k in range(_ROW // _LANES):
            j = g * (_ROW // _LANES) + k
            s = pl.ds(j * _LANES, _LANES)
            idx_v[s] = idx_v[s] + pats[(j * _LANES) % _N_FIELDS]

    def gather(g, b):
        return pltpu.async_copy(
            table_hbm.at[idx_v.at[pl.ds(g * _ROW, _ROW)]], rows_v.at[b],
            gsems[b])

    def write(g, b):
        return pltpu.async_copy(
            rows_v.at[b], out_hbm.at[pl.ds(base + g * _ROW, _ROW)], wsems[b])

    hg = [None] * _NB
    hw = [None] * _NB
    for b in range(_NB):
        add_offsets(b)
        hg[b] = gather(b, b)
    for g in range(_NB, _G):
        add_offsets(g)
    for g in range(_G):
        b = g % _NB
        hg[b].wait()
        hw[b] = write(g, b)
        nxt = g + _NB
        if nxt < _G:
            hw[b].wait()
            hg[b] = gather(nxt, b)
    for g in range(max(0, _G - _NB), _G):
        hw[g % _NB].wait()


def kernel(x, table):
    out = _emb_lookup(x.reshape(_TOTAL), table)
    return out.reshape(_BATCH, _N_FIELDS, _EMBED_DIM)


# speedup vs baseline: 1.5950x; 1.0031x over previous
import functools

import jax
import jax.numpy as jnp
from jax import lax
from jax.experimental import pallas as pl
from jax.experimental.pallas import tpu as pltpu
from jax.experimental.pallas import tpu_sc as plsc

_N_FIELDS = 26
_EMBED_DIM = 64
_BATCH = 4096
_TOTAL = _BATCH * _N_FIELDS
_NC, _NS, _LANES = 2, 16, 16
_NW = _NC * _NS
_PER_W = _TOTAL // _NW
_ROW = 128
_G = _PER_W // _ROW
_PAIRS = _G // 2
_NP = 3

_mesh = plsc.VectorSubcoreMesh(core_axis_name="c", subcore_axis_name="s")


@functools.partial(
    pl.kernel,
    mesh=_mesh,
    out_type=jax.ShapeDtypeStruct((_TOTAL, _EMBED_DIM), jnp.float32),
    scratch_types=[
        pltpu.VMEM((_PER_W,), jnp.int32),
        pltpu.VMEM((_NP, 2 * _ROW, _EMBED_DIM), jnp.float32),
        [pltpu.SemaphoreType.DMA] * (2 * _NP),
        [pltpu.SemaphoreType.DMA] * _NP,
    ],
    compiler_params=pltpu.CompilerParams(use_tc_tiling_on_sc=False),
)
def _emb_lookup(x_hbm, table_hbm, out_hbm, idx_v, rows_v, gsems, wsems):
    wid = lax.axis_index("s") * _NC + lax.axis_index("c")
    base = wid * _PER_W

    lane = lax.iota(jnp.int32, _LANES)
    pats = {
        s: ((s + lane) % _N_FIELDS) * 1000 for s in range(0, _N_FIELDS, 2)
    }

    pltpu.sync_copy(x_hbm.at[pl.ds(base, _PER_W)], idx_v)

    def add_offsets(g):
        for k in range(_ROW // _LANES):
            j = g * (_ROW // _LANES) + k
            s = pl.ds(j * _LANES, _LANES)
            idx_v[s] = idx_v[s] + pats[(j * _LANES) % _N_FIELDS]

    def gather(g, p, h):
        return pltpu.async_copy(
            table_hbm.at[idx_v.at[pl.ds(g * _ROW, _ROW)]],
            rows_v.at[p].at[pl.ds(h * _ROW, _ROW)],
            gsems[2 * p + h])

    def write(q, p):
        return pltpu.async_copy(
            rows_v.at[p], out_hbm.at[pl.ds(base + q * 2 * _ROW, 2 * _ROW)],
            wsems[p])

    hg = [[None, None] for _ in range(_NP)]
    hw = [None] * _NP
    for q in range(_NP):
        add_offsets(2 * q)
        add_offsets(2 * q + 1)
        hg[q] = [gather(2 * q, q, 0), gather(2 * q + 1, q, 1)]
    for g in range(2 * _NP, _G):
        add_offsets(g)
    for q in range(_PAIRS):
        p = q % _NP
        hg[p][0].wait()
        hg[p][1].wait()
        hw[p] = write(q, p)
        nq = q + _NP
        if nq < _PAIRS:
            hw[p].wait()
            hg[p] = [gather(2 * nq, p, 0), gather(2 * nq + 1, p, 1)]
    for q in range(max(0, _PAIRS - _NP), _PAIRS):
        hw[q % _NP].wait()


def kernel(x, table):
    out = _emb_lookup(x.reshape(_TOTAL), table)
    return out.reshape(_BATCH, _N_FIELDS, _EMBED_DIM)
